# baseline (device time: 23553 ns/iter reference)
import jax
import jax.numpy as jnp
from jax import lax
from jax.experimental import pallas as pl
from jax.experimental.pallas import tpu as pltpu

N_DEV = 8
NR = 5
NE = 4


def kernel(x, w_mat):
    m_per, k = x.shape
    _, n_per = w_mat.shape
    p = m_per // 2

    def body(x_ref, w_ref, out_ref, xg_ref, w_vmem, out_vmem,
             fs, fr, bs, br, es, er, cp_sems, out_sems):
        l = lax.axis_index("i")
        pos = jnp.where(l < 4, l, 11 - l)
        parity = lax.rem(pos, 2)
        s = 1 - 2 * parity

        def l_of(qq):
            qq = lax.rem(qq + 2 * N_DEV, N_DEV)
            return jnp.where(qq < 4, qq, 11 - qq)

        right = l_of(pos + 1)
        left = l_of(pos - 1)
        prt = l_of(pos + 3 * s)

        def o(kk):
            return l_of(pos - kk)

        def q_(kk):
            return l_of(pos + kk)

        def piece(pidx):
            return xg_ref.at[pl.ds(pidx * p, p), :]

        def send(src, pidx, dev, ssem, rsem):
            d = pltpu.make_async_remote_copy(
                src_ref=src, dst_ref=piece(pidx),
                send_sem=ssem, recv_sem=rsem,
                device_id=(dev,), device_id_type=pl.DeviceIdType.MESH,
            )
            d.start()
            return d

        def recv_wait(pidx, ssem, rsem):
            pltpu.make_async_remote_copy(
                src_ref=piece(pidx), dst_ref=piece(pidx),
                send_sem=ssem, recv_sem=rsem,
                device_id=(right,), device_id_type=pl.DeviceIdType.MESH,
            ).wait_recv()

        out_cps = []

        def flush_out(row0, nrows):
            d = pltpu.make_async_copy(
                out_vmem.at[pl.ds(row0, nrows), :],
                out_ref.at[pl.ds(row0, nrows), :],
                out_sems.at[len(out_cps)],
            )
            d.start()
            out_cps.append(d)

        def gemm_piece(pidx):
            out_vmem[pl.ds(pidx * p, p), :] = jnp.maximum(
                jnp.dot(piece(pidx)[...], w_vmem[...],
                        preferred_element_type=jnp.float32),
                0.0,
            )
            flush_out(pidx * p, p)

        x_cp = pltpu.make_async_copy(
            x_ref, xg_ref.at[pl.ds(l * m_per, m_per), :], cp_sems.at[0])
        x_cp.start()
        w_cp = pltpu.make_async_copy(w_ref, w_vmem, cp_sems.at[1])
        w_cp.start()

        barrier_sem = pltpu.get_barrier_semaphore()
        for nbr in (left, right, prt):
            pl.semaphore_signal(
                barrier_sem, inc=1,
                device_id=(nbr,), device_id_type=pl.DeviceIdType.MESH,
            )
        pl.semaphore_wait(barrier_sem, 3)

        fsendp = [2 * o(0), 2 * o(0) + 1, 2 * o(1), 2 * o(1) + 1, 2 * o(2)]
        frecvp = [2 * o(1), 2 * o(1) + 1, 2 * o(2), 2 * o(2) + 1, 2 * o(3)]
        bsendp = [2 * q_(0) + 1, 2 * q_(0), 2 * q_(1) + 1, 2 * q_(1),
                  2 * q_(2) + 1]
        brecvp = [2 * q_(1) + 1, 2 * q_(1), 2 * q_(2) + 1, 2 * q_(2),
                  2 * q_(3) + 1]
        c1 = l_of(pos - s)
        c2 = l_of(pos - 2 * s)
        esendp = [2 * l + 1 - parity,
                  2 * c1 + parity,
                  2 * c1 + 1 - parity,
                  2 * c2 + parity]
        a = l_of(pos + 4)
        erecvp = [2 * prt + parity,
                  2 * a + 1 - parity,
                  2 * a + parity,
                  2 * l_of(pos - 3 * s) + 1 - parity]

        started = []
        x_cp.wait()
        started.append(send(piece(fsendp[0]), fsendp[0], right,
                            fs.at[0], fr.at[0]))
        started.append(send(piece(fsendp[1]), fsendp[1], right,
                            fs.at[1], fr.at[1]))
        started.append(send(piece(bsendp[0]), bsendp[0], left,
                            bs.at[0], br.at[0]))
        started.append(send(piece(bsendp[1]), bsendp[1], left,
                            bs.at[1], br.at[1]))
        started.append(send(piece(esendp[0]), esendp[0], prt,
                            es.at[0], er.at[0]))

        w_cp.wait()
        out_vmem[pl.ds(l * m_per, m_per), :] = jnp.maximum(
            jnp.dot(xg_ref[pl.ds(l * m_per, m_per), :], w_vmem[...],
                    preferred_element_type=jnp.float32),
            0.0,
        )
        flush_out(l * m_per, m_per)

        for j in range(NR):
            recv_wait(frecvp[j], fs.at[j], fr.at[j])
            if j + 2 < NR:
                started.append(send(piece(fsendp[j + 2]), fsendp[j + 2],
                                    right, fs.at[j + 2], fr.at[j + 2]))
            recv_wait(brecvp[j], bs.at[j], br.at[j])
            if j + 2 < NR:
                started.append(send(piece(bsendp[j + 2]), bsendp[j + 2],
                                    left, bs.at[j + 2], br.at[j + 2]))
            if j < 3:
                started.append(send(piece(esendp[j + 1]), esendp[j + 1],
                                    prt, es.at[j + 1], er.at[j + 1]))
            gemm_piece(frecvp[j])
            gemm_piece(brecvp[j])
            t = [0, None, 1, 2, 3][j]
            if t is not None:
                recv_wait(erecvp[t], es.at[t], er.at[t])
                gemm_piece(erecvp[t])

        for d in started:
            d.wait_send()
        for d in out_cps:
            d.wait()

    return pl.pallas_call(
        body,
        out_shape=jax.ShapeDtypeStruct((N_DEV * m_per, n_per), jnp.float32),
        in_specs=[
            pl.BlockSpec(memory_space=pltpu.MemorySpace.HBM),
            pl.BlockSpec(memory_space=pltpu.MemorySpace.HBM),
        ],
        out_specs=pl.BlockSpec(memory_space=pltpu.MemorySpace.HBM),
        scratch_shapes=[
            pltpu.VMEM((N_DEV * m_per, k), jnp.float32),
            pltpu.VMEM((k, n_per), jnp.float32),
            pltpu.VMEM((N_DEV * m_per, n_per), jnp.float32),
            pltpu.SemaphoreType.DMA((NR,)),
            pltpu.SemaphoreType.DMA((NR,)),
            pltpu.SemaphoreType.DMA((NR,)),
            pltpu.SemaphoreType.DMA((NR,)),
            pltpu.SemaphoreType.DMA((NE,)),
            pltpu.SemaphoreType.DMA((NE,)),
            pltpu.SemaphoreType.DMA((2,)),
            pltpu.SemaphoreType.DMA((15,)),
        ],
        compiler_params=pltpu.CompilerParams(collective_id=0),
    )(x, w_mat)
